# Initial kernel scaffold; baseline (speedup 1.0000x reference)
import jax
import jax.numpy as jnp
from jax.experimental import pallas as pl


def kernel(indices, tables, W1, b1, W2, b2):
    raise NotImplementedError("write your pallas kernel here")


# speedup vs baseline: 7.4224x; 7.4224x over previous
import functools

import jax
import jax.numpy as jnp
from jax import lax
from jax.experimental import pallas as pl
from jax.experimental.pallas import tpu as pltpu
from jax.experimental.pallas import tpu_sc as plsc

F = 26
V = 100000
D = 32
B = 16384
LOG2_B = 14

NC = 2
NS = 16
NW = NC * NS
TOTAL_ROWS = F * B
RPW = TOTAL_ROWS // NW
CHUNK = 1664
NCHUNK = RPW // CHUNK


def _sc_gather_body(tables_hbm, idx_hbm, out_hbm,
                    idx_a, idx_b, rows_a, rows_b, sem_a, sem_b):
    wid = lax.axis_index("s") * NC + lax.axis_index("c")
    base = pl.multiple_of(wid * RPW, CHUNK)
    idx_bufs = (idx_a, idx_b)
    row_bufs = (rows_a, rows_b)
    sems = (sem_a, sem_b)

    def stage_indices(c, k):
        pltpu.sync_copy(idx_hbm.at[pl.ds(base + c * CHUNK, CHUNK)], idx_bufs[k])

        def fix(i, _):
            off = pl.multiple_of(i * 16, 16)
            pos = base + c * CHUNK + off + lax.iota(jnp.int32, 16)
            fld = lax.shift_right_logical(pos, LOG2_B)
            idx_bufs[k][pl.ds(off, 16)] = idx_bufs[k][pl.ds(off, 16)] + fld * V
            return 0

        lax.fori_loop(0, CHUNK // 16, fix, 0)

    prev = None
    for c in range(NCHUNK):
        k = c % 2
        stage_indices(c, k)
        cp = pltpu.async_copy(tables_hbm.at[idx_bufs[k]], row_bufs[k], sems[k])
        if prev is not None:
            prev.wait()
            pltpu.sync_copy(row_bufs[1 - k],
                            out_hbm.at[pl.ds(base + (c - 1) * CHUNK, CHUNK)])
        prev = cp
    prev.wait()
    last = NCHUNK - 1
    pltpu.sync_copy(row_bufs[last % 2],
                    out_hbm.at[pl.ds(base + last * CHUNK, CHUNK)])


@functools.lru_cache(maxsize=None)
def _sc_gather():
    return pl.kernel(
        _sc_gather_body,
        mesh=plsc.VectorSubcoreMesh(core_axis_name="c", subcore_axis_name="s"),
        out_type=jax.ShapeDtypeStruct((TOTAL_ROWS, D), jnp.float32),
        scratch_types=[
            pltpu.VMEM((CHUNK,), jnp.int32),
            pltpu.VMEM((CHUNK,), jnp.int32),
            pltpu.VMEM((CHUNK, D), jnp.float32),
            pltpu.VMEM((CHUNK, D), jnp.float32),
            pltpu.SemaphoreType.DMA,
            pltpu.SemaphoreType.DMA,
        ],
        compiler_params=pltpu.CompilerParams(use_tc_tiling_on_sc=False),
    )


BT = 2048


def _mlp_body(embs_ref, w1_ref, b1_ref, w2_ref, b2_ref, out_ref):
    acc = jnp.zeros((BT, D), jnp.float32)
    for f in range(F):
        acc = acc + jnp.dot(embs_ref[f], w1_ref[f],
                            preferred_element_type=jnp.float32)
    h = jnp.maximum(acc + b1_ref[...], 0.0)
    out_ref[...] = jnp.dot(h, w2_ref[...],
                           preferred_element_type=jnp.float32) + b2_ref[...]


def _mlp(embs_fbd, W1r, b1_2d, W2, b2_2d):
    return pl.pallas_call(
        _mlp_body,
        grid=(B // BT,),
        in_specs=[
            pl.BlockSpec((F, BT, D), lambda i: (0, i, 0)),
            pl.BlockSpec((F, D, D), lambda i: (0, 0, 0)),
            pl.BlockSpec((1, D), lambda i: (0, 0)),
            pl.BlockSpec((D, D), lambda i: (0, 0)),
            pl.BlockSpec((1, D), lambda i: (0, 0)),
        ],
        out_specs=pl.BlockSpec((BT, D), lambda i: (i, 0)),
        out_shape=jax.ShapeDtypeStruct((B, D), jnp.float32),
    )(embs_fbd, W1r, b1_2d, W2, b2_2d)


def kernel(indices, tables, W1, b1, W2, b2):
    tables_flat = tables.reshape(F * V, D)
    idx_flat = indices.reshape(TOTAL_ROWS)
    embs = _sc_gather()(tables_flat, idx_flat)
    embs_fbd = embs.reshape(F, B, D)
    return _mlp(embs_fbd, W1.reshape(F, D, D), b1.reshape(1, D),
                W2, b2.reshape(1, D))
